# baseline (speedup 1.0000x reference)
import functools

import jax
import jax.numpy as jnp
from jax import lax
from jax.experimental import pallas as pl
from jax.experimental.pallas import tpu as pltpu
from jax.experimental.pallas import tpu_sc as plsc

_B1, _B2, _V = 32, 8, 128256
_R = _B1 * _B2
_NW = 32
_ROWS_PER_W = _R // _NW
_L = 16
_NSTEP = _V // _L
_UNROLL = 16
_OUTER = _NSTEP // _UNROLL
_INT_MAX = 2**31 - 1

_mesh = plsc.VectorSubcoreMesh(
    core_axis_name="c", subcore_axis_name="s", num_cores=2, num_subcores=16)


def _argmax_rows_body(x_hbm, out_hbm, row_v, res_v):
    wid = lax.axis_index("s") * 2 + lax.axis_index("c")
    iota = lax.iota(jnp.int32, _L)
    res = jnp.zeros((_L,), jnp.int32)
    for r in range(_ROWS_PER_W):
        row = wid * _ROWS_PER_W + r
        pltpu.sync_copy(x_hbm.at[row], row_v)

        def body(i, carry):
            vmax, vidx = carry
            base = i * (_UNROLL * _L)
            for j in range(_UNROLL):
                v = row_v[pl.ds(base + j * _L, _L)]
                idx = iota + (base + j * _L)
                m = v > vmax
                vmax = jnp.where(m, v, vmax)
                vidx = jnp.where(m, idx, vidx)
            return vmax, vidx

        init = (
            jnp.full((_L,), -jnp.inf, jnp.float32),
            jnp.zeros((_L,), jnp.int32),
        )
        vmax, vidx = lax.fori_loop(0, _OUTER, body, init)
        for off in (8, 4, 2, 1):
            perm = iota ^ off
            v2 = vmax.at[perm].get(mode="promise_in_bounds")
            i2 = vidx.at[perm].get(mode="promise_in_bounds")
            better = (v2 > vmax) | ((v2 == vmax) & (i2 < vidx))
            vmax = jnp.where(better, v2, vmax)
            vidx = jnp.where(better, i2, vidx)
        res = jnp.where(iota == r, vidx, res)
    res_v[...] = res
    base_out = pl.multiple_of(wid * _ROWS_PER_W, 8)
    pltpu.sync_copy(res_v.at[pl.ds(0, _ROWS_PER_W)],
                    out_hbm.at[pl.ds(base_out, _ROWS_PER_W)])


_argmax_rows = functools.partial(
    pl.kernel,
    mesh=_mesh,
    out_type=jax.ShapeDtypeStruct((_R,), jnp.int32),
    scratch_types=[
        pltpu.VMEM((_V,), jnp.float32),
        pltpu.VMEM((_L,), jnp.int32),
    ],
)(_argmax_rows_body)


def kernel(logits):
    flat = logits.reshape(_R, _V)
    out = _argmax_rows(flat)
    return out.reshape(_B1, _B2)
